# baseline (speedup 1.0000x reference)
import functools

import jax
import jax.numpy as jnp
from jax import lax
from jax.experimental import pallas as pl
from jax.experimental.pallas import tpu as pltpu
from jax.experimental.pallas import tpu_sc as plsc

N = 10000
NPAD = 10240
IN_CH = 128
HID = 16
HEADS1 = 8
HEADS2 = 1
WALK_WINDOW = 5
NEG_SAMPLES = 10
TEMP = 0.07
NWALK = 16
LWALK = 20
PAD_K = 16

NC = 2
NS = 16
C = 112
NSLOT = 2
NQ = 6
DLAG = NSLOT - 1
NCH0 = 96
NCH1 = 96
E_PAD = NS * (NCH0 + NCH1) * C
RPT = NPAD // NS
BLK = 1280

_sc_mesh = plsc.VectorSubcoreMesh(
    core_axis_name="c", subcore_axis_name="s", num_cores=NC, num_subcores=NS)


def _make_conv_edge(heads):
  def body(h_hbm, at_hbm, bt_hbm, sd_hbm, z128, z16, out_hbm, den_hbm, *scr):
    c = lax.axis_index("c")
    t = lax.axis_index("s")
    it = iter(scr)
    sd = tuple(next(it) for _ in range(NQ))
    ea = tuple(next(it) for _ in range(NSLOT))
    eb = tuple(next(it) for _ in range(NSLOT))
    hb = tuple(next(it) for _ in range(NSLOT))
    isem = tuple(next(it) for _ in range(NQ))
    gsem = tuple(next(it) for _ in range(NSLOT))
    ssem = tuple(next(it) for _ in range(NSLOT))
    acc_sp = next(it)
    den_sp = next(it)

    pltpu.sync_copy(z128, acc_sp.at[pl.ds(t * RPT, RPT)])
    pltpu.sync_copy(z16, den_sp.at[pl.ds(t * RPT, RPT)])
    plsc.subcore_barrier()

    nchunk = jnp.where(c == 0, NCH0, NCH1)
    cbase = jnp.where(c == 0, t * NCH0, NS * NCH0 + t * NCH1)

    def fetch_idx(cj, q):
      pltpu.async_copy(sd_hbm.at[cbase + cj], sd[q], isem[q])

    def wait_idx(q):
      pltpu.make_async_copy(sd_hbm.at[0], sd[q], isem[q]).wait()

    def gathers(s, q):
      pltpu.async_copy(h_hbm.at[sd[q].at[0]], hb[s], gsem[s])
      pltpu.async_copy(at_hbm.at[sd[q].at[0]], ea[s], gsem[s])
      pltpu.async_copy(bt_hbm.at[sd[q].at[1]], eb[s], gsem[s])

    def wait_gathers(s, q):
      pltpu.make_async_copy(h_hbm.at[sd[q].at[0]], hb[s], gsem[s]).wait()
      pltpu.make_async_copy(at_hbm.at[sd[q].at[0]], ea[s], gsem[s]).wait()
      pltpu.make_async_copy(bt_hbm.at[sd[q].at[1]], eb[s], gsem[s]).wait()

    def wait_scatter(s, q):
      pltpu.make_async_copy(hb[s], acc_sp.at[sd[q].at[1]], ssem[s]).wait()
      pltpu.make_async_copy(ea[s], den_sp.at[sd[q].at[1]], ssem[s]).wait()

    def compute(s):
      def ebody(e, carry):
        v = ea[s][e, :] + eb[s][e, :]
        v = jnp.maximum(v, 0.2 * v)
        v = jnp.exp(v)
        ea[s][e, :] = v
        for g in range(8):
          hb[s][e, pl.ds(16 * g, 16)] = hb[s][e, pl.ds(16 * g, 16)] * v[g]
        return carry
      lax.fori_loop(0, C, ebody, 0)

    def scatter(s, q):
      pltpu.async_copy(hb[s], acc_sp.at[sd[q].at[1]], ssem[s], add=True)
      pltpu.async_copy(ea[s], den_sp.at[sd[q].at[1]], ssem[s], add=True)

    fetch_idx(0, 0)
    fetch_idx(1, 1)
    wait_idx(0)
    gathers(0, 0)

    def step(cj, s, q):
      @pl.when(cj >= DLAG)
      def _():
        wait_scatter((s + 1) % NSLOT, (q + NQ - DLAG) % NQ)

      @pl.when(cj + 2 < nchunk)
      def _():
        fetch_idx(cj + 2, (q + 2) % NQ)

      @pl.when(cj + 1 < nchunk)
      def _():
        wait_idx((q + 1) % NQ)
        gathers((s + 1) % NSLOT, (q + 1) % NQ)

      wait_gathers(s, q)
      compute(s)
      scatter(s, q)

    def outer(i, carry):
      for b in range(NQ):
        step(i * NQ + b, b % NSLOT, b)
      return carry
    lax.fori_loop(0, nchunk // NQ, outer, 0)

    for k in range(DLAG):
      cj = NCH0 - DLAG + k
      wait_scatter(cj % NSLOT, cj % NQ)
    plsc.subcore_barrier()
    pltpu.sync_copy(acc_sp.at[pl.ds(t * RPT, RPT)],
                    out_hbm.at[c, pl.ds(t * RPT, RPT)])
    pltpu.sync_copy(den_sp.at[pl.ds(t * RPT, RPT)],
                    den_hbm.at[c, pl.ds(t * RPT, RPT)])

  idx_t = lambda: pltpu.VMEM((2, C), jnp.int32)
  e_t = lambda: pltpu.VMEM((C, 16), jnp.float32)
  h_t = lambda: pltpu.VMEM((C, 128), jnp.float32)
  sem = pltpu.SemaphoreType.DMA
  return pl.kernel(
      body,
      out_type=(jax.ShapeDtypeStruct((NC, NPAD, 128), jnp.float32),
                jax.ShapeDtypeStruct((NC, NPAD, 16), jnp.float32)),
      mesh=_sc_mesh,
      compiler_params=pltpu.CompilerParams(use_tc_tiling_on_sc=False),
      scratch_types=(
          tuple(idx_t() for _ in range(NQ))
          + tuple(e_t() for _ in range(2 * NSLOT))
          + tuple(h_t() for _ in range(NSLOT))
          + tuple(sem for _ in range(NQ + 2 * NSLOT))
          + (pltpu.VMEM_SHARED((NPAD, 128), jnp.float32),
             pltpu.VMEM_SHARED((NPAD, 16), jnp.float32))
      ),
  )


_conv_edge = _make_conv_edge(8)

NROWS = 3 * NWALK * LWALK * PAD_K
RW = NROWS // (NC * NS)
GCH = 120


def _gather_body(emb_hbm, gidx_hbm, g_hbm, gi, rb):
  wid = lax.axis_index("c") * NS + lax.axis_index("s")
  base = wid * RW
  for k in range(RW // GCH):
    b = base + k * GCH
    pltpu.sync_copy(gidx_hbm.at[pl.ds(b, GCH)], gi)
    pltpu.sync_copy(emb_hbm.at[gi], rb)
    pltpu.sync_copy(rb, g_hbm.at[pl.ds(b, GCH)])


_loss_gather = pl.kernel(
    _gather_body,
    out_type=jax.ShapeDtypeStruct((NROWS, 256), jnp.float32),
    mesh=_sc_mesh,
    scratch_types=(
        pltpu.VMEM((GCH,), jnp.int32),
        pltpu.VMEM((GCH, 256), jnp.float32),
    ),
)


def _prep1_kernel(x_ref, w_ref, pas_ref, pad_ref, h_out, as_out, ad_out):
  h = jnp.dot(x_ref[...], w_ref[...], preferred_element_type=jnp.float32)
  h_out[...] = h
  as_out[...] = jnp.dot(h, pas_ref[...], preferred_element_type=jnp.float32)
  ad_out[...] = jnp.dot(h, pad_ref[...], preferred_element_type=jnp.float32)


def _mid_kernel(p0, p1, d0, d1, exp_ref, b_ref, w2_ref, pas_ref, pad_ref,
                emb1_out, h_out, as_out, ad_out):
  den = jnp.dot(d0[...] + d1[...], exp_ref[...],
                preferred_element_type=jnp.float32)
  agg = (p0[...] + p1[...]) / (den + 1e-16) + b_ref[...]
  e1 = jnp.where(agg > 0, agg, jnp.exp(agg) - 1.0)
  emb1_out[...] = e1
  h2 = jnp.dot(e1, w2_ref[...], preferred_element_type=jnp.float32)
  h_out[...] = h2
  as_out[...] = jnp.dot(h2, pas_ref[...], preferred_element_type=jnp.float32)
  ad_out[...] = jnp.dot(h2, pad_ref[...], preferred_element_type=jnp.float32)


def _final_kernel(e1_ref, q0, q1, d0, d1, exp_ref, b_ref, emb_out):
  den = jnp.dot(d0[...] + d1[...], exp_ref[...],
                preferred_element_type=jnp.float32)
  e2 = (q0[...] + q1[...]) / (den + 1e-16) + b_ref[...]
  emb_out[:, :128] = e1_ref[...]
  emb_out[:, 128:] = e2


def _loss_kernel(ar_ref, pr_ref, nr_ref, pm_ref, km_ref, out_ref):
  AR = ar_ref[...]
  PR = pr_ref[...]
  NR = nr_ref[...]
  pm = pm_ref[...]
  km = km_ref[...]
  inv_a = 1.0 / jnp.maximum(jnp.sqrt(jnp.sum(AR * AR, axis=-1)), 1e-8)
  inv_p = 1.0 / jnp.maximum(jnp.sqrt(jnp.sum(PR * PR, axis=-1)), 1e-8)
  inv_n = 1.0 / jnp.maximum(jnp.sqrt(jnp.sum(NR * NR, axis=-1)), 1e-8)
  dots_p = jnp.sum(AR * PR, axis=-1) * inv_a * inv_p * (1.0 / TEMP)
  dots_n = jnp.sum(AR * NR, axis=-1) * inv_a * inv_n * (1.0 / TEMP)
  pos_sum = jnp.sum(jnp.exp(dots_p) * pm, axis=-1)
  neg_sum = jnp.sum(jnp.exp(dots_n) * km, axis=-1)
  terms = jnp.log(pos_sum + neg_sum) - jnp.log(pos_sum)
  out_ref[...] = jnp.sum(terms).reshape(1, 1)


def _window_map():
  import numpy as np
  posmap = np.zeros((LWALK, PAD_K), dtype=np.int32)
  valid = np.zeros((LWALK, PAD_K), dtype=np.float32)
  for i in range(LWALK):
    js = [j for j in range(i - WALK_WINDOW, i + WALK_WINDOW + 1)
          if j != i and 0 <= j < LWALK]
    for k, j in enumerate(js):
      posmap[i, k] = j
      valid[i, k] = 1.0
  return jnp.asarray(posmap), jnp.asarray(valid)


def _neg_indices(n):
  base = jax.random.key(1234)
  wi = jnp.arange(NWALK, dtype=jnp.int32)
  ii = jnp.arange(LWALK, dtype=jnp.int32)

  def one(w, i):
    k = jax.random.fold_in(jax.random.fold_in(base, w), i)
    return jax.random.randint(k, (NEG_SAMPLES,), 0, n)

  return jax.vmap(lambda w: jax.vmap(lambda i: one(w, i))(ii))(wi)


def _grid_call(fn, n_out_128, outs, *args):
  grid = NPAD // BLK

  def spec(arr):
    k = arr.shape[-1]
    if arr.shape[0] == NPAD:
      return pl.BlockSpec((BLK, k), lambda i: (i, 0))
    return pl.BlockSpec(arr.shape, lambda i: (0, 0))

  return pl.pallas_call(
      fn,
      grid=(grid,),
      in_specs=[spec(a) for a in args],
      out_specs=[pl.BlockSpec((BLK, k), lambda i: (i, 0)) for k in outs],
      out_shape=[jax.ShapeDtypeStruct((NPAD, k), jnp.float32) for k in outs],
  )(*args)


def kernel(x, edge_index, walks, W1, a1_src, a1_dst, b1, W2, a2_src, a2_dst, b2):
  n = x.shape[0]
  loops = jnp.arange(n, dtype=edge_index.dtype)
  pad = jnp.full((E_PAD - n - edge_index.shape[1],), N, dtype=edge_index.dtype)
  src = jnp.concatenate([edge_index[0], loops, pad])
  dst = jnp.concatenate([edge_index[1], loops, pad])
  tot_chunks = E_PAD // C
  sd = jnp.stack([src.reshape(tot_chunks, C), dst.reshape(tot_chunks, C)],
                 axis=1)
  x_pad = jnp.pad(x, ((0, NPAD - n), (0, 0)))

  eye8 = jnp.eye(8, dtype=jnp.float32)
  A1s = jnp.pad((a1_src[:, :, None] * eye8[:, None, :]).reshape(128, 8),
                ((0, 0), (0, 8)))
  A1d = jnp.pad((a1_dst[:, :, None] * eye8[:, None, :]).reshape(128, 8),
                ((0, 0), (0, 8)))
  rep8 = jnp.concatenate([jnp.ones((1, 8), jnp.float32),
                          jnp.zeros((1, 8), jnp.float32)], axis=1)
  A2s = a2_src.reshape(128, 1) * rep8
  A2d = a2_dst.reshape(128, 1) * rep8
  EXP16 = jnp.pad((eye8[:, :, None] * jnp.ones((16,), jnp.float32))
                  .reshape(8, 128), ((0, 8), (0, 0)))
  EXP1 = jnp.zeros((16, 128), jnp.float32).at[0, :].set(1.0)
  z128 = jnp.zeros((RPT, 128), jnp.float32)
  z16 = jnp.zeros((RPT, 16), jnp.float32)

  h1, as1, ad1 = _grid_call(_prep1_kernel, None, (128, 16, 16),
                            x_pad, W1, A1s, A1d)
  out1, den1 = _conv_edge(h1, as1, ad1, sd, z128, z16)
  emb1, h2, as2, ad2 = _grid_call(
      _mid_kernel, None, (128, 128, 16, 16),
      out1[0], out1[1], den1[0], den1[1], EXP16, b1.reshape(1, 128),
      W2, A2s, A2d)
  out2, den2 = _conv_edge(h2, as2, ad2, sd, z128, z16)
  emb = _grid_call(_final_kernel, None, (256,),
                   emb1, out2[0], out2[1], den2[0], den2[1], EXP1,
                   b2.reshape(1, 128))[0]

  posmap, pvalid = _window_map()
  A = NWALK * LWALK
  anchor_idx = walks.reshape(-1)
  pos_idx = jnp.take(walks, posmap.reshape(-1), axis=1).reshape(A, PAD_K)
  pmask = jnp.tile(pvalid, (NWALK, 1))
  neg = _neg_indices(n).reshape(A, NEG_SAMPLES)
  coll = (neg[:, :, None] == pos_idx[:, None, :]) & (pmask[:, None, :] > 0)
  keep = (~coll.any(-1)).astype(jnp.float32)
  neg_idx = jnp.pad(neg, ((0, 0), (0, PAD_K - NEG_SAMPLES)))
  kmask = jnp.pad(keep, ((0, 0), (0, PAD_K - NEG_SAMPLES)))

  gidx = jnp.concatenate([
      jnp.repeat(anchor_idx, PAD_K),
      pos_idx.reshape(-1),
      neg_idx.reshape(-1),
  ]).astype(jnp.int32)
  G = _loss_gather(emb, gidx)
  AP = A * PAD_K
  AR = G[:AP].reshape(A, PAD_K, 256)
  PR = G[AP:2 * AP].reshape(A, PAD_K, 256)
  NR = G[2 * AP:].reshape(A, PAD_K, 256)

  out = pl.pallas_call(
      _loss_kernel,
      out_shape=jax.ShapeDtypeStruct((1, 1), jnp.float32),
  )(AR, PR, NR, pmask, kmask)
  return out.reshape(())


# speedup vs baseline: 1.2891x; 1.2891x over previous
import functools

import jax
import jax.numpy as jnp
from jax import lax
from jax.experimental import pallas as pl
from jax.experimental.pallas import tpu as pltpu
from jax.experimental.pallas import tpu_sc as plsc

N = 10000
NPAD = 10240
IN_CH = 128
HID = 16
HEADS1 = 8
HEADS2 = 1
WALK_WINDOW = 5
NEG_SAMPLES = 10
TEMP = 0.07
NWALK = 16
LWALK = 20
PAD_K = 16

NC = 2
NS = 16
C = 64
NCHUNK = 168
PER_TILE = NCHUNK * C
E_HALF = NS * PER_TILE
E_PAD = NC * E_HALF
RPT = NPAD // NS
BLK = 1280

_sc_mesh = plsc.VectorSubcoreMesh(
    core_axis_name="c", subcore_axis_name="s", num_cores=NC, num_subcores=NS)


def _make_conv_edge(heads):
  def body(h_hbm, at_hbm, bt_hbm, src_hbm, dst_hbm, z128, z16,
           out_hbm, den_hbm, *scr):
    c = lax.axis_index("c")
    t = lax.axis_index("s")
    it = iter(scr)
    sidx = tuple(next(it) for _ in range(3))
    didx = tuple(next(it) for _ in range(3))
    ea = tuple(next(it) for _ in range(3))
    eb = tuple(next(it) for _ in range(3))
    hb = tuple(next(it) for _ in range(3))
    ex = tuple(next(it) for _ in range(3))
    gsem = tuple(next(it) for _ in range(3))
    ssem = tuple(next(it) for _ in range(3))
    out_sp = next(it)
    den_sp = next(it)

    pltpu.sync_copy(z128, out_sp.at[pl.ds(t * RPT, RPT)])
    pltpu.sync_copy(z16, den_sp.at[pl.ds(t * RPT, RPT)])
    plsc.subcore_barrier()

    ebase = c * E_HALF + t * PER_TILE

    def issue(cj, s):
      b = ebase + cj * C
      pltpu.sync_copy(src_hbm.at[pl.ds(b, C)], sidx[s])
      pltpu.sync_copy(dst_hbm.at[pl.ds(b, C)], didx[s])
      pltpu.async_copy(at_hbm.at[sidx[s]], ea[s], gsem[s])
      pltpu.async_copy(bt_hbm.at[didx[s]], eb[s], gsem[s])
      pltpu.async_copy(h_hbm.at[sidx[s]], hb[s], gsem[s])

    def wait_gathers(s):
      pltpu.make_async_copy(at_hbm.at[sidx[s]], ea[s], gsem[s]).wait()
      pltpu.make_async_copy(bt_hbm.at[didx[s]], eb[s], gsem[s]).wait()
      pltpu.make_async_copy(h_hbm.at[sidx[s]], hb[s], gsem[s]).wait()

    def wait_scatters(s):
      pltpu.make_async_copy(ex[s], den_sp.at[didx[s]], ssem[s]).wait()
      pltpu.make_async_copy(hb[s], out_sp.at[didx[s]], ssem[s]).wait()

    def compute(s):
      def ebody(e, carry):
        v = ea[s][e, :] + eb[s][e, :]
        v = jnp.maximum(v, 0.2 * v)
        v = jnp.exp(v)
        ex[s][e, :] = v
        for g in range(8):
          hb[s][e, pl.ds(16 * g, 16)] = hb[s][e, pl.ds(16 * g, 16)] * v[g]
        return carry
      lax.fori_loop(0, C, ebody, 0)

    def scatters(s):
      pltpu.async_copy(ex[s], den_sp.at[didx[s]], ssem[s], add=True)
      pltpu.async_copy(hb[s], out_sp.at[didx[s]], ssem[s], add=True)

    issue(0, 0)

    def step(cj, s):
      @pl.when(jnp.logical_and(cj + 1 < NCHUNK, cj >= 2))
      def _():
        wait_scatters((s + 1) % 3)

      @pl.when(cj + 1 < NCHUNK)
      def _():
        issue(cj + 1, (s + 1) % 3)

      wait_gathers(s)
      compute(s)
      scatters(s)

    def outer(i, carry):
      for b in range(3):
        step(i * 3 + b, b)
      return carry
    lax.fori_loop(0, NCHUNK // 3, outer, 0)

    wait_scatters(0)
    wait_scatters(1)
    wait_scatters(2)
    plsc.subcore_barrier()
    pltpu.sync_copy(out_sp.at[pl.ds(t * RPT, RPT)],
                    out_hbm.at[c, pl.ds(t * RPT, RPT)])
    pltpu.sync_copy(den_sp.at[pl.ds(t * RPT, RPT)],
                    den_hbm.at[c, pl.ds(t * RPT, RPT)])

  idx_t = lambda: pltpu.VMEM((C,), jnp.int32)
  e_t = lambda: pltpu.VMEM((C, 16), jnp.float32)
  h_t = lambda: pltpu.VMEM((C, 128), jnp.float32)
  sem = pltpu.SemaphoreType.DMA
  return pl.kernel(
      body,
      out_type=(jax.ShapeDtypeStruct((NC, NPAD, 128), jnp.float32),
                jax.ShapeDtypeStruct((NC, NPAD, 16), jnp.float32)),
      mesh=_sc_mesh,
      compiler_params=pltpu.CompilerParams(use_tc_tiling_on_sc=False),
      scratch_types=(
          tuple(idx_t() for _ in range(6))
          + tuple(e_t() for _ in range(6))
          + tuple(h_t() for _ in range(3))
          + tuple(e_t() for _ in range(3))
          + tuple(sem for _ in range(6))
          + (pltpu.VMEM_SHARED((NPAD, 128), jnp.float32),
             pltpu.VMEM_SHARED((NPAD, 16), jnp.float32))
      ),
  )


_conv_edge = _make_conv_edge(8)

NROWS = 3 * NWALK * LWALK * PAD_K
RW = NROWS // (NC * NS)
GCH = 120


def _gather_body(emb_hbm, gidx_hbm, g_hbm, gi, rb):
  wid = lax.axis_index("c") * NS + lax.axis_index("s")
  base = wid * RW
  for k in range(RW // GCH):
    b = base + k * GCH
    pltpu.sync_copy(gidx_hbm.at[pl.ds(b, GCH)], gi)
    pltpu.sync_copy(emb_hbm.at[gi], rb)
    pltpu.sync_copy(rb, g_hbm.at[pl.ds(b, GCH)])


_loss_gather = pl.kernel(
    _gather_body,
    out_type=jax.ShapeDtypeStruct((NROWS, 256), jnp.float32),
    mesh=_sc_mesh,
    scratch_types=(
        pltpu.VMEM((GCH,), jnp.int32),
        pltpu.VMEM((GCH, 256), jnp.float32),
    ),
)


def _prep1_kernel(x_ref, w_ref, pas_ref, pad_ref, h_out, as_out, ad_out):
  h = jnp.dot(x_ref[...], w_ref[...], preferred_element_type=jnp.float32)
  h_out[...] = h
  as_out[...] = jnp.dot(h, pas_ref[...], preferred_element_type=jnp.float32)
  ad_out[...] = jnp.dot(h, pad_ref[...], preferred_element_type=jnp.float32)


def _mid_kernel(p0, p1, d0, d1, exp_ref, b_ref, w2_ref, pas_ref, pad_ref,
                emb1_out, h_out, as_out, ad_out):
  den = jnp.dot(d0[...] + d1[...], exp_ref[...],
                preferred_element_type=jnp.float32)
  agg = (p0[...] + p1[...]) / (den + 1e-16) + b_ref[...]
  e1 = jnp.where(agg > 0, agg, jnp.exp(agg) - 1.0)
  emb1_out[...] = e1
  h2 = jnp.dot(e1, w2_ref[...], preferred_element_type=jnp.float32)
  h_out[...] = h2
  as_out[...] = jnp.dot(h2, pas_ref[...], preferred_element_type=jnp.float32)
  ad_out[...] = jnp.dot(h2, pad_ref[...], preferred_element_type=jnp.float32)


def _final_kernel(e1_ref, q0, q1, d0, d1, exp_ref, b_ref, emb_out):
  den = jnp.dot(d0[...] + d1[...], exp_ref[...],
                preferred_element_type=jnp.float32)
  e2 = (q0[...] + q1[...]) / (den + 1e-16) + b_ref[...]
  emb_out[:, :128] = e1_ref[...]
  emb_out[:, 128:] = e2


def _loss_kernel(ar_ref, pr_ref, nr_ref, pm_ref, km_ref, out_ref):
  AR = ar_ref[...]
  PR = pr_ref[...]
  NR = nr_ref[...]
  pm = pm_ref[...]
  km = km_ref[...]
  inv_a = 1.0 / jnp.maximum(jnp.sqrt(jnp.sum(AR * AR, axis=-1)), 1e-8)
  inv_p = 1.0 / jnp.maximum(jnp.sqrt(jnp.sum(PR * PR, axis=-1)), 1e-8)
  inv_n = 1.0 / jnp.maximum(jnp.sqrt(jnp.sum(NR * NR, axis=-1)), 1e-8)
  dots_p = jnp.sum(AR * PR, axis=-1) * inv_a * inv_p * (1.0 / TEMP)
  dots_n = jnp.sum(AR * NR, axis=-1) * inv_a * inv_n * (1.0 / TEMP)
  pos_sum = jnp.sum(jnp.exp(dots_p) * pm, axis=-1)
  neg_sum = jnp.sum(jnp.exp(dots_n) * km, axis=-1)
  terms = jnp.log(pos_sum + neg_sum) - jnp.log(pos_sum)
  out_ref[...] = jnp.sum(terms).reshape(1, 1)


def _window_map():
  import numpy as np
  posmap = np.zeros((LWALK, PAD_K), dtype=np.int32)
  valid = np.zeros((LWALK, PAD_K), dtype=np.float32)
  for i in range(LWALK):
    js = [j for j in range(i - WALK_WINDOW, i + WALK_WINDOW + 1)
          if j != i and 0 <= j < LWALK]
    for k, j in enumerate(js):
      posmap[i, k] = j
      valid[i, k] = 1.0
  return jnp.asarray(posmap), jnp.asarray(valid)


def _neg_indices(n):
  base = jax.random.key(1234)
  wi = jnp.arange(NWALK, dtype=jnp.int32)
  ii = jnp.arange(LWALK, dtype=jnp.int32)

  def one(w, i):
    k = jax.random.fold_in(jax.random.fold_in(base, w), i)
    return jax.random.randint(k, (NEG_SAMPLES,), 0, n)

  return jax.vmap(lambda w: jax.vmap(lambda i: one(w, i))(ii))(wi)


def _grid_call(fn, n_out_128, outs, *args):
  grid = NPAD // BLK

  def spec(arr):
    k = arr.shape[-1]
    if arr.shape[0] == NPAD:
      return pl.BlockSpec((BLK, k), lambda i: (i, 0))
    return pl.BlockSpec(arr.shape, lambda i: (0, 0))

  return pl.pallas_call(
      fn,
      grid=(grid,),
      in_specs=[spec(a) for a in args],
      out_specs=[pl.BlockSpec((BLK, k), lambda i: (i, 0)) for k in outs],
      out_shape=[jax.ShapeDtypeStruct((NPAD, k), jnp.float32) for k in outs],
  )(*args)


def kernel(x, edge_index, walks, W1, a1_src, a1_dst, b1, W2, a2_src, a2_dst, b2):
  n = x.shape[0]
  loops = jnp.arange(n, dtype=edge_index.dtype)
  pad = jnp.full((E_PAD - n - edge_index.shape[1],), N, dtype=edge_index.dtype)
  src = jnp.concatenate([edge_index[0], loops, pad])
  dst = jnp.concatenate([edge_index[1], loops, pad])
  x_pad = jnp.pad(x, ((0, NPAD - n), (0, 0)))

  eye8 = jnp.eye(8, dtype=jnp.float32)
  A1s = jnp.pad((a1_src[:, :, None] * eye8[:, None, :]).reshape(128, 8),
                ((0, 0), (0, 8)))
  A1d = jnp.pad((a1_dst[:, :, None] * eye8[:, None, :]).reshape(128, 8),
                ((0, 0), (0, 8)))
  rep8 = jnp.concatenate([jnp.ones((1, 8), jnp.float32),
                          jnp.zeros((1, 8), jnp.float32)], axis=1)
  A2s = a2_src.reshape(128, 1) * rep8
  A2d = a2_dst.reshape(128, 1) * rep8
  EXP16 = jnp.pad((eye8[:, :, None] * jnp.ones((16,), jnp.float32))
                  .reshape(8, 128), ((0, 8), (0, 0)))
  EXP1 = jnp.zeros((16, 128), jnp.float32).at[0, :].set(1.0)
  z128 = jnp.zeros((RPT, 128), jnp.float32)
  z16 = jnp.zeros((RPT, 16), jnp.float32)

  h1, as1, ad1 = _grid_call(_prep1_kernel, None, (128, 16, 16),
                            x_pad, W1, A1s, A1d)
  out1, den1 = _conv_edge(h1, as1, ad1, src, dst, z128, z16)
  emb1, h2, as2, ad2 = _grid_call(
      _mid_kernel, None, (128, 128, 16, 16),
      out1[0], out1[1], den1[0], den1[1], EXP16, b1.reshape(1, 128),
      W2, A2s, A2d)
  out2, den2 = _conv_edge(h2, as2, ad2, src, dst, z128, z16)
  emb = _grid_call(_final_kernel, None, (256,),
                   emb1, out2[0], out2[1], den2[0], den2[1], EXP1,
                   b2.reshape(1, 128))[0]

  posmap, pvalid = _window_map()
  A = NWALK * LWALK
  anchor_idx = walks.reshape(-1)
  pos_idx = jnp.take(walks, posmap.reshape(-1), axis=1).reshape(A, PAD_K)
  pmask = jnp.tile(pvalid, (NWALK, 1))
  neg = _neg_indices(n).reshape(A, NEG_SAMPLES)
  coll = (neg[:, :, None] == pos_idx[:, None, :]) & (pmask[:, None, :] > 0)
  keep = (~coll.any(-1)).astype(jnp.float32)
  neg_idx = jnp.pad(neg, ((0, 0), (0, PAD_K - NEG_SAMPLES)))
  kmask = jnp.pad(keep, ((0, 0), (0, PAD_K - NEG_SAMPLES)))

  gidx = jnp.concatenate([
      jnp.repeat(anchor_idx, PAD_K),
      pos_idx.reshape(-1),
      neg_idx.reshape(-1),
  ]).astype(jnp.int32)
  G = _loss_gather(emb, gidx)
  AP = A * PAD_K
  AR = G[:AP].reshape(A, PAD_K, 256)
  PR = G[AP:2 * AP].reshape(A, PAD_K, 256)
  NR = G[2 * AP:].reshape(A, PAD_K, 256)

  out = pl.pallas_call(
      _loss_kernel,
      out_shape=jax.ShapeDtypeStruct((1, 1), jnp.float32),
  )(AR, PR, NR, pmask, kmask)
  return out.reshape(())
